# baseline (speedup 1.0000x reference)
import functools

import jax
import jax.numpy as jnp
from jax import lax
from jax.experimental import pallas as pl
from jax.experimental.pallas import tpu as pltpu
from jax.experimental.pallas import tpu_sc as plsc

EMBED_DIM = 32
BATCH = 16384
TILE = 128

NUM_CORES = 2
NUM_SUBCORES = 16
NUM_WORKERS = NUM_CORES * NUM_SUBCORES
LANES = 16
HALF = LANES
SROW = LANES + 1
DEPTH = 4

SC_SHARE = 12288
TC_SHARE = BATCH - SC_SHARE
TC_K = 8

B_PER_W = SC_SHARE // NUM_WORKERS
GROUPS = B_PER_W // LANES


def _sc_body(user_hbm, item_hbm, u_hbm, i_hbm, out_hbm,
             u_vm, i_vm, ub0, ub1, ub2, ub3, ib0, ib1, ib2, ib3,
             psum, out_v, su0, su1, su2, su3, si0, si1, si2, si3):
  wid = lax.axis_index("s") * NUM_CORES + lax.axis_index("c")
  base = wid * B_PER_W

  pltpu.sync_copy(u_hbm.at[pl.ds(base, B_PER_W)], u_vm)
  pltpu.sync_copy(i_hbm.at[pl.ds(base, B_PER_W)], i_vm)

  iota = lax.iota(jnp.int32, LANES)
  ubufs = (ub0, ub1, ub2, ub3)
  ibufs = (ib0, ib1, ib2, ib3)
  usems = (su0, su1, su2, su3)
  isems = (si0, si1, si2, si3)

  def fire(j, k, slot):
    cj = pl.multiple_of((j >> 7) * TILE, TILE)
    pltpu.async_copy(
        user_hbm.at[:, pl.ds(cj, TILE)], ubufs[slot], usems[slot])
    ck = pl.multiple_of((k >> 7) * TILE, TILE)
    pltpu.async_copy(
        item_hbm.at[:, pl.ds(ck, TILE)], ibufs[slot], isems[slot])

  def consume(j, k, s, slot):
    pltpu.make_async_copy(
        user_hbm.at[:, pl.ds(0, TILE)], ubufs[slot], usems[slot]).wait()
    pltpu.make_async_copy(
        item_hbm.at[:, pl.ds(0, TILE)], ibufs[slot], isems[slot]).wait()
    lu_v = jnp.full((LANES,), j & (TILE - 1), jnp.int32)
    li_v = jnp.full((LANES,), k & (TILE - 1), jnp.int32)
    u0 = plsc.load_gather(ubufs[slot], [iota, lu_v])
    u1 = plsc.load_gather(ubufs[slot], [iota + HALF, lu_v])
    v0 = plsc.load_gather(ibufs[slot], [iota, li_v])
    v1 = plsc.load_gather(ibufs[slot], [iota + HALF, li_v])
    psum[s, pl.ds(0, LANES)] = u0 * v0 + u1 * v1

  u16p = u_vm[pl.ds(0, LANES)]
  i16p = i_vm[pl.ds(0, LANES)]
  for s in range(DEPTH - 1):
    fire(u16p[s], i16p[s], s % DEPTH)

  @pl.loop(0, GROUPS)
  def _group(g):
    u16 = u_vm[pl.ds(g * LANES, LANES)]
    i16 = i_vm[pl.ds(g * LANES, LANES)]
    js = [u16[s] for s in range(LANES)]
    ks = [i16[s] for s in range(LANES)]
    for s in range(LANES):
      if s + DEPTH - 1 < LANES:
        fire(js[s + DEPTH - 1], ks[s + DEPTH - 1], (s + DEPTH - 1) % DEPTH)
      else:
        lane = s + DEPTH - 1 - LANES

        @pl.when(g < GROUPS - 1)
        def _():
          u16n = u_vm[pl.ds((g + 1) * LANES, LANES)]
          i16n = i_vm[pl.ds((g + 1) * LANES, LANES)]
          fire(u16n[lane], i16n[lane], (s + DEPTH - 1) % DEPTH)

      consume(js[s], ks[s], s, s % DEPTH)
    acc = plsc.load_gather(psum, [iota, jnp.zeros((LANES,), jnp.int32)])
    for col in range(1, LANES):
      acc = acc + plsc.load_gather(
          psum, [iota, jnp.full((LANES,), col, jnp.int32)])
    out_v[pl.ds(g * LANES, LANES)] = acc

  pltpu.sync_copy(out_v, out_hbm.at[pl.ds(base, B_PER_W)])


@functools.partial(
    pl.kernel,
    out_type=jax.ShapeDtypeStruct((SC_SHARE,), jnp.float32),
    mesh=plsc.VectorSubcoreMesh(
        core_axis_name="c", subcore_axis_name="s",
        num_cores=NUM_CORES, num_subcores=NUM_SUBCORES),
    compiler_params=pltpu.CompilerParams(
        needs_layout_passes=False, use_tc_tiling_on_sc=True),
    scratch_types=[
        pltpu.VMEM((B_PER_W,), jnp.int32),
        pltpu.VMEM((B_PER_W,), jnp.int32),
    ] + [pltpu.VMEM((EMBED_DIM, TILE), jnp.float32)] * 8 + [
        pltpu.VMEM((LANES, SROW), jnp.float32),
        pltpu.VMEM((B_PER_W,), jnp.float32),
    ] + [pltpu.SemaphoreType.DMA] * 8,
)
def _sc_dot(user_hbm, item_hbm, u_hbm, i_hbm, out_hbm, *rest):
  _sc_body(user_hbm, item_hbm, u_hbm, i_hbm, out_hbm, *rest)


def _tc_kernel(u_sref, i_sref, *refs):
  slab_refs, out_ref = refs[:-1], refs[-1]
  u_slabs = slab_refs[:TC_K]
  i_slabs = slab_refs[TC_K:]
  b = pl.program_id(0)
  lane = lax.broadcasted_iota(jnp.int32, (1, TILE), 1)
  dots = []
  for t in range(TC_K):
    lu = u_sref[b * TC_K + t] & (TILE - 1)
    li = i_sref[b * TC_K + t] & (TILE - 1)
    cu = jnp.sum(jnp.where(lane == lu, u_slabs[t][...], 0.0),
                 axis=1, keepdims=True)
    ci = jnp.sum(jnp.where(lane == li, i_slabs[t][...], 0.0),
                 axis=1, keepdims=True)
    dots.append(jnp.sum(cu * ci))
  out_ref[b % 8, :] = jnp.stack(dots)


def _make_tc():
  def u_map(t):
    return lambda b, u_sref, i_sref: (0, u_sref[b * TC_K + t] >> 7)

  def i_map(t):
    return lambda b, u_sref, i_sref: (0, i_sref[b * TC_K + t] >> 7)

  grid_spec = pltpu.PrefetchScalarGridSpec(
      num_scalar_prefetch=2,
      grid=(TC_SHARE // TC_K,),
      in_specs=(
          [pl.BlockSpec((EMBED_DIM, TILE), u_map(t)) for t in range(TC_K)]
          + [pl.BlockSpec((EMBED_DIM, TILE), i_map(t)) for t in range(TC_K)]
      ),
      out_specs=pl.BlockSpec((8, TC_K), lambda b, u_sref, i_sref: (b // 8, 0)),
  )
  return pl.pallas_call(
      _tc_kernel,
      grid_spec=grid_spec,
      out_shape=jax.ShapeDtypeStruct((TC_SHARE // TC_K, TC_K), jnp.float32),
  )


_tc_dot = _make_tc()


def kernel(u, i, user_table, item_table):
  u32 = u.astype(jnp.int32)
  i32 = i.astype(jnp.int32)
  ut = user_table.T
  it = item_table.T
  out_sc = _sc_dot(ut, it, u32[:SC_SHARE], i32[:SC_SHARE])
  tc_args = [ut] * TC_K + [it] * TC_K
  out_tc = _tc_dot(u32[SC_SHARE:], i32[SC_SHARE:], *tc_args)
  return jnp.concatenate([out_sc, out_tc.reshape((TC_SHARE,))])


# speedup vs baseline: 1.8354x; 1.8354x over previous
import functools

import jax
import jax.numpy as jnp
from jax import lax
from jax.experimental import pallas as pl
from jax.experimental.pallas import tpu as pltpu
from jax.experimental.pallas import tpu_sc as plsc

EMBED_DIM = 32
BATCH = 16384
TILE = 128

NUM_CORES = 2
NUM_SUBCORES = 16
NUM_WORKERS = NUM_CORES * NUM_SUBCORES
B_PER_W = BATCH // NUM_WORKERS
LANES = 16
HALF = LANES
GROUPS = B_PER_W // LANES
SROW = LANES + 1
DEPTH = 4


def _body(user_hbm, item_hbm, u_hbm, i_hbm, out_hbm,
          u_vm, i_vm, ub0, ub1, ub2, ub3, ib0, ib1, ib2, ib3, psum, out_v,
          su0, su1, su2, su3, si0, si1, si2, si3):
  wid = lax.axis_index("s") * NUM_CORES + lax.axis_index("c")
  base = wid * B_PER_W

  pltpu.sync_copy(u_hbm.at[pl.ds(base, B_PER_W)], u_vm)
  pltpu.sync_copy(i_hbm.at[pl.ds(base, B_PER_W)], i_vm)

  iota = lax.iota(jnp.int32, LANES)
  ubufs = (ub0, ub1, ub2, ub3)
  ibufs = (ib0, ib1, ib2, ib3)
  usems = (su0, su1, su2, su3)
  isems = (si0, si1, si2, si3)

  def fire(j, k, slot):
    cj = pl.multiple_of((j >> 7) * TILE, TILE)
    pltpu.async_copy(
        user_hbm.at[:, pl.ds(cj, TILE)], ubufs[slot], usems[slot])
    ck = pl.multiple_of((k >> 7) * TILE, TILE)
    pltpu.async_copy(
        item_hbm.at[:, pl.ds(ck, TILE)], ibufs[slot], isems[slot])

  def consume(j, k, s, slot):
    pltpu.make_async_copy(
        user_hbm.at[:, pl.ds(0, TILE)], ubufs[slot], usems[slot]).wait()
    pltpu.make_async_copy(
        item_hbm.at[:, pl.ds(0, TILE)], ibufs[slot], isems[slot]).wait()
    lu_v = jnp.full((LANES,), j & (TILE - 1), jnp.int32)
    li_v = jnp.full((LANES,), k & (TILE - 1), jnp.int32)
    u0 = plsc.load_gather(ubufs[slot], [iota, lu_v])
    u1 = plsc.load_gather(ubufs[slot], [iota + HALF, lu_v])
    v0 = plsc.load_gather(ibufs[slot], [iota, li_v])
    v1 = plsc.load_gather(ibufs[slot], [iota + HALF, li_v])
    psum[s, pl.ds(0, LANES)] = u0 * v0 + u1 * v1

  u16p = u_vm[pl.ds(0, LANES)]
  i16p = i_vm[pl.ds(0, LANES)]
  for s in range(DEPTH - 1):
    fire(u16p[s], i16p[s], s % DEPTH)

  @pl.loop(0, GROUPS)
  def _group(g):
    u16 = u_vm[pl.ds(g * LANES, LANES)]
    i16 = i_vm[pl.ds(g * LANES, LANES)]
    js = [u16[s] for s in range(LANES)]
    ks = [i16[s] for s in range(LANES)]
    for s in range(LANES):
      if s + DEPTH - 1 < LANES:
        fire(js[s + DEPTH - 1], ks[s + DEPTH - 1], (s + DEPTH - 1) % DEPTH)
      else:
        lane = s + DEPTH - 1 - LANES

        @pl.when(g < GROUPS - 1)
        def _():
          u16n = u_vm[pl.ds((g + 1) * LANES, LANES)]
          i16n = i_vm[pl.ds((g + 1) * LANES, LANES)]
          fire(u16n[lane], i16n[lane], (s + DEPTH - 1) % DEPTH)

      consume(js[s], ks[s], s, s % DEPTH)
    acc = plsc.load_gather(psum, [iota, jnp.zeros((LANES,), jnp.int32)])
    for col in range(1, LANES):
      acc = acc + plsc.load_gather(
          psum, [iota, jnp.full((LANES,), col, jnp.int32)])
    out_v[pl.ds(g * LANES, LANES)] = acc

  pltpu.sync_copy(out_v, out_hbm.at[pl.ds(base, B_PER_W)])


@functools.partial(
    pl.kernel,
    out_type=jax.ShapeDtypeStruct((BATCH,), jnp.float32),
    mesh=plsc.VectorSubcoreMesh(
        core_axis_name="c", subcore_axis_name="s",
        num_cores=NUM_CORES, num_subcores=NUM_SUBCORES),
    compiler_params=pltpu.CompilerParams(
        needs_layout_passes=False, use_tc_tiling_on_sc=True),
    scratch_types=[
        pltpu.VMEM((B_PER_W,), jnp.int32),
        pltpu.VMEM((B_PER_W,), jnp.int32),
    ] + [pltpu.VMEM((EMBED_DIM, TILE), jnp.float32)] * 8 + [
        pltpu.VMEM((LANES, SROW), jnp.float32),
        pltpu.VMEM((B_PER_W,), jnp.float32),
    ] + [pltpu.SemaphoreType.DMA] * 8,
)
def _sc_dot(user_hbm, item_hbm, u_hbm, i_hbm, out_hbm,
            u_vm, i_vm, ub0, ub1, ub2, ub3, ib0, ib1, ib2, ib3, psum, out_v,
            su0, su1, su2, su3, si0, si1, si2, si3):
  _body(user_hbm, item_hbm, u_hbm, i_hbm, out_hbm,
        u_vm, i_vm, ub0, ub1, ub2, ub3, ib0, ib1, ib2, ib3, psum, out_v,
        su0, su1, su2, su3, si0, si1, si2, si3)


def kernel(u, i, user_table, item_table):
  return _sc_dot(user_table.T, item_table.T,
                 u.astype(jnp.int32), i.astype(jnp.int32))
